# initial kernel scaffold (unmeasured)
import functools

import jax
import jax.numpy as jnp
from jax import lax
from jax.experimental import pallas as pl
from jax.experimental.pallas import tpu as pltpu


def _mm_allreduce_body(
    x_ref, w_ref, out_ref, send_buf, recv_buf, send_sem, recv_sem, *, axis, relu
):
    my_x = lax.axis_index("x")
    my_y = lax.axis_index("y")
    if axis == "y":
        target = (my_x, 1 - my_y)
    else:
        target = (1 - my_x, my_y)

    barrier = pltpu.get_barrier_semaphore()
    pl.semaphore_signal(
        barrier, inc=1, device_id=target, device_id_type=pl.DeviceIdType.MESH
    )
    pl.semaphore_wait(barrier, 1)

    send_buf[...] = jnp.dot(
        x_ref[...], w_ref[...], preferred_element_type=jnp.float32
    )

    rdma = pltpu.make_async_remote_copy(
        src_ref=send_buf,
        dst_ref=recv_buf,
        send_sem=send_sem,
        recv_sem=recv_sem,
        device_id=target,
        device_id_type=pl.DeviceIdType.MESH,
    )
    rdma.start()
    rdma.wait()

    acc = send_buf[...] + recv_buf[...]
    if relu:
        acc = jnp.maximum(acc, 0.0)
    out_ref[...] = acc


def _mm_allreduce(x, w, *, axis, relu, collective_id):
    m, _ = x.shape
    _, n = w.shape
    body = functools.partial(_mm_allreduce_body, axis=axis, relu=relu)
    return pl.pallas_call(
        body,
        out_shape=jax.ShapeDtypeStruct((m, n), jnp.float32),
        in_specs=[
            pl.BlockSpec(memory_space=pltpu.VMEM),
            pl.BlockSpec(memory_space=pltpu.VMEM),
        ],
        out_specs=pl.BlockSpec(memory_space=pltpu.VMEM),
        scratch_shapes=[
            pltpu.VMEM((m, n), jnp.float32),
            pltpu.VMEM((m, n), jnp.float32),
            pltpu.SemaphoreType.DMA,
            pltpu.SemaphoreType.DMA,
        ],
        compiler_params=pltpu.CompilerParams(collective_id=collective_id),
    )(x, w)


def kernel(x, Win0, Wout0, Win1, Wout1, Win2, Wout2):
    for i, (win, wout) in enumerate(
        [(Win0, Wout0), (Win1, Wout1), (Win2, Wout2)]
    ):
        h = _mm_allreduce(x, win, axis="y", relu=True, collective_id=2 * i)
        x = _mm_allreduce(h, wout, axis="x", relu=False, collective_id=2 * i + 1)
    return x


# baseline (device time: 158795 ns/iter reference)
import functools

import jax
import jax.numpy as jnp
from jax import lax
from jax.experimental import pallas as pl
from jax.experimental.pallas import tpu as pltpu


def _mm_allreduce_body(
    x_ref, w_ref, out_ref, send_buf, recv_buf, send_sem, recv_sem, *, axis, relu
):
    my_x = lax.axis_index("x")
    my_y = lax.axis_index("y")
    if axis == "y":
        target = (my_x, 1 - my_y)
    else:
        target = (1 - my_x, my_y)

    barrier = pltpu.get_barrier_semaphore()
    pl.semaphore_signal(
        barrier, inc=1, device_id=target, device_id_type=pl.DeviceIdType.MESH
    )
    pl.semaphore_wait(barrier, 1)

    send_buf[...] = jnp.dot(
        x_ref[...], w_ref[...], preferred_element_type=jnp.float32
    )

    rdma = pltpu.make_async_remote_copy(
        src_ref=send_buf,
        dst_ref=recv_buf,
        send_sem=send_sem,
        recv_sem=recv_sem,
        device_id=target,
        device_id_type=pl.DeviceIdType.MESH,
    )
    rdma.start()
    rdma.wait()

    acc = send_buf[...] + recv_buf[...]
    if relu:
        acc = jnp.maximum(acc, 0.0)
    out_ref[...] = acc


def _mm_allreduce(x, w, *, axis, relu, collective_id):
    m, _ = x.shape
    _, n = w.shape
    body = functools.partial(_mm_allreduce_body, axis=axis, relu=relu)
    return pl.pallas_call(
        body,
        out_shape=jax.ShapeDtypeStruct((m, n), jnp.float32),
        in_specs=[
            pl.BlockSpec(memory_space=pltpu.VMEM),
            pl.BlockSpec(memory_space=pltpu.VMEM),
        ],
        out_specs=pl.BlockSpec(memory_space=pltpu.VMEM),
        scratch_shapes=[
            pltpu.VMEM((m, n), jnp.float32),
            pltpu.VMEM((m, n), jnp.float32),
            pltpu.SemaphoreType.DMA,
            pltpu.SemaphoreType.DMA,
        ],
        compiler_params=pltpu.CompilerParams(
            collective_id=collective_id,
            vmem_limit_bytes=60 * 1024 * 1024,
        ),
    )(x, w)


def kernel(x, Win0, Wout0, Win1, Wout1, Win2, Wout2):
    for i, (win, wout) in enumerate(
        [(Win0, Wout0), (Win1, Wout1), (Win2, Wout2)]
    ):
        h = _mm_allreduce(x, win, axis="y", relu=True, collective_id=2 * i)
        x = _mm_allreduce(h, wout, axis="x", relu=False, collective_id=2 * i + 1)
    return x


# device time: 106012 ns/iter; 1.4979x vs baseline; 1.4979x over previous
import functools

import jax
import jax.numpy as jnp
from jax import lax
from jax.experimental import pallas as pl
from jax.experimental.pallas import tpu as pltpu

NC = 8
CH = 512


def _layer_body(
    x_ref, win_ref, wout_ref, out_ref,
    wa, wb, h_send, h_recv, o_send, o_recv,
    wa_sem, wb_sem, hs_sems, hr_sems, os_sem, or_sem,
):
    my_x = lax.axis_index("x")
    my_y = lax.axis_index("y")
    y_peer = (my_x, 1 - my_y)
    x_peer = (1 - my_x, my_y)

    barrier = pltpu.get_barrier_semaphore()
    for t in (y_peer, x_peer):
        pl.semaphore_signal(
            barrier, inc=1, device_id=t, device_id_type=pl.DeviceIdType.MESH
        )
    pl.semaphore_wait(barrier, 2)

    def win_dma(c):
        return pltpu.make_async_copy(
            win_ref.at[:, pl.ds(c * CH, CH)], wa.at[c % 2], wa_sem.at[c % 2]
        )

    def wout_dma(c):
        return pltpu.make_async_copy(
            wout_ref.at[pl.ds(c * CH, CH), :], wb.at[c % 2], wb_sem.at[c % 2]
        )

    def h_rdma(c):
        return pltpu.make_async_remote_copy(
            src_ref=h_send.at[c],
            dst_ref=h_recv.at[c],
            send_sem=hs_sems.at[c],
            recv_sem=hr_sems.at[c],
            device_id=y_peer,
            device_id_type=pl.DeviceIdType.MESH,
        )

    win_dma(0).start()
    win_dma(1).start()
    wout_dma(0).start()
    wout_dma(1).start()

    def produce(c):
        win_dma(c).wait()
        h_send[c] = jnp.dot(
            x_ref[...], wa[c % 2], preferred_element_type=jnp.float32
        )
        h_rdma(c).start()
        if c + 2 < NC:
            win_dma(c + 2).start()

    def consume(c):
        wout_dma(c).wait()
        h_rdma(c).wait_recv()
        hc = jnp.maximum(h_send[c] + h_recv[c], 0.0)
        inc = jnp.dot(hc, wb[c % 2], preferred_element_type=jnp.float32)
        if c == 0:
            o_send[...] = inc
        else:
            o_send[...] = o_send[...] + inc
        if c + 2 < NC:
            wout_dma(c + 2).start()

    produce(0)
    for c in range(NC):
        if c + 1 < NC:
            produce(c + 1)
        consume(c)

    o_rdma = pltpu.make_async_remote_copy(
        src_ref=o_send,
        dst_ref=o_recv,
        send_sem=os_sem,
        recv_sem=or_sem,
        device_id=x_peer,
        device_id_type=pl.DeviceIdType.MESH,
    )
    o_rdma.start()
    o_rdma.wait()
    out_ref[...] = o_send[...] + o_recv[...]

    for c in range(NC):
        h_rdma(c).wait_send()


def _layer(x, win, wout, *, collective_id):
    m, k = x.shape
    _, n = win.shape
    assert n == NC * CH and wout.shape == (n, k)
    return pl.pallas_call(
        _layer_body,
        out_shape=jax.ShapeDtypeStruct((m, k), jnp.float32),
        in_specs=[
            pl.BlockSpec(memory_space=pltpu.VMEM),
            pl.BlockSpec(memory_space=pl.ANY),
            pl.BlockSpec(memory_space=pl.ANY),
        ],
        out_specs=pl.BlockSpec(memory_space=pltpu.VMEM),
        scratch_shapes=[
            pltpu.VMEM((2, k, CH), jnp.float32),
            pltpu.VMEM((2, CH, k), jnp.float32),
            pltpu.VMEM((NC, m, CH), jnp.float32),
            pltpu.VMEM((NC, m, CH), jnp.float32),
            pltpu.VMEM((m, k), jnp.float32),
            pltpu.VMEM((m, k), jnp.float32),
            pltpu.SemaphoreType.DMA((2,)),
            pltpu.SemaphoreType.DMA((2,)),
            pltpu.SemaphoreType.DMA((NC,)),
            pltpu.SemaphoreType.DMA((NC,)),
            pltpu.SemaphoreType.DMA,
            pltpu.SemaphoreType.DMA,
        ],
        compiler_params=pltpu.CompilerParams(
            collective_id=collective_id,
            vmem_limit_bytes=60 * 1024 * 1024,
        ),
    )(x, win, wout)


def kernel(x, Win0, Wout0, Win1, Wout1, Win2, Wout2):
    for i, (win, wout) in enumerate(
        [(Win0, Wout0), (Win1, Wout1), (Win2, Wout2)]
    ):
        x = _layer(x, win, wout, collective_id=i)
    return x


# device time: 95749 ns/iter; 1.6585x vs baseline; 1.1072x over previous
import jax
import jax.numpy as jnp
from jax import lax
from jax.experimental import pallas as pl
from jax.experimental.pallas import tpu as pltpu

N_LAYERS = 3
NC = 8
CH = 512
NGC = N_LAYERS * NC
LAG = 2


def _body(
    x_ref, win0, wout0, win1, wout1, win2, wout2, out_ref,
    wa, wb, h_send, h_recv, o_send, o_recv, x_buf,
    wa_sem, wb_sem, hs_sems, hr_sems, os_sems, or_sems,
):
    my_x = lax.axis_index("x")
    my_y = lax.axis_index("y")
    y_peer = (my_x, 1 - my_y)
    x_peer = (1 - my_x, my_y)
    wins = [win0, win1, win2]
    wouts = [wout0, wout1, wout2]

    barrier = pltpu.get_barrier_semaphore()
    for t in (y_peer, x_peer):
        pl.semaphore_signal(
            barrier, inc=1, device_id=t, device_id_type=pl.DeviceIdType.MESH
        )
    pl.semaphore_wait(barrier, 2)

    def win_dma(gc):
        l, c = divmod(gc, NC)
        return pltpu.make_async_copy(
            wins[l].at[:, pl.ds(c * CH, CH)], wa.at[gc % 2], wa_sem.at[gc % 2]
        )

    def wout_dma(gc):
        l, c = divmod(gc, NC)
        return pltpu.make_async_copy(
            wouts[l].at[pl.ds(c * CH, CH), :], wb.at[gc % 2], wb_sem.at[gc % 2]
        )

    def h_rdma(gc):
        return pltpu.make_async_remote_copy(
            src_ref=h_send.at[gc],
            dst_ref=h_recv.at[gc],
            send_sem=hs_sems.at[gc],
            recv_sem=hr_sems.at[gc],
            device_id=y_peer,
            device_id_type=pl.DeviceIdType.MESH,
        )

    def produce(gc):
        l, _ = divmod(gc, NC)
        src = x_ref if l == 0 else x_buf
        win_dma(gc).wait()
        h_send[gc] = jnp.dot(
            src[...], wa[gc % 2], preferred_element_type=jnp.float32
        )
        h_rdma(gc).start()
        if gc + 2 < NGC:
            win_dma(gc + 2).start()

    def consume(gc):
        l, c = divmod(gc, NC)
        wout_dma(gc).wait()
        h_rdma(gc).wait_recv()
        hc = jnp.maximum(h_send[gc] + h_recv[gc], 0.0)
        inc = jnp.dot(hc, wb[gc % 2], preferred_element_type=jnp.float32)
        if c == 0:
            o_send[l] = inc
        else:
            o_send[l] = o_send[l] + inc
        if gc + 2 < NGC:
            wout_dma(gc + 2).start()

    def layer_end(l):
        o_rdma = pltpu.make_async_remote_copy(
            src_ref=o_send.at[l],
            dst_ref=o_recv.at[l],
            send_sem=os_sems.at[l],
            recv_sem=or_sems.at[l],
            device_id=x_peer,
            device_id_type=pl.DeviceIdType.MESH,
        )
        o_rdma.start()
        o_rdma.wait()
        dst = out_ref if l == N_LAYERS - 1 else x_buf
        dst[...] = o_send[l] + o_recv[l]

    win_dma(0).start()
    win_dma(1).start()
    wout_dma(0).start()
    wout_dma(1).start()

    for l in range(N_LAYERS):
        base = l * NC
        for c in range(LAG):
            produce(base + c)
        for c in range(NC):
            if c + LAG < NC:
                produce(base + c + LAG)
            consume(base + c)
        layer_end(l)

    for gc in range(NGC):
        h_rdma(gc).wait_send()


def kernel(x, Win0, Wout0, Win1, Wout1, Win2, Wout2):
    m, k = x.shape
    any_spec = pl.BlockSpec(memory_space=pl.ANY)
    return pl.pallas_call(
        _body,
        out_shape=jax.ShapeDtypeStruct((m, k), jnp.float32),
        in_specs=[pl.BlockSpec(memory_space=pltpu.VMEM)] + [any_spec] * 6,
        out_specs=pl.BlockSpec(memory_space=pltpu.VMEM),
        scratch_shapes=[
            pltpu.VMEM((2, k, CH), jnp.float32),
            pltpu.VMEM((2, CH, k), jnp.float32),
            pltpu.VMEM((NGC, m, CH), jnp.float32),
            pltpu.VMEM((NGC, m, CH), jnp.float32),
            pltpu.VMEM((N_LAYERS, m, k), jnp.float32),
            pltpu.VMEM((N_LAYERS, m, k), jnp.float32),
            pltpu.VMEM((m, k), jnp.float32),
            pltpu.SemaphoreType.DMA((2,)),
            pltpu.SemaphoreType.DMA((2,)),
            pltpu.SemaphoreType.DMA((NGC,)),
            pltpu.SemaphoreType.DMA((NGC,)),
            pltpu.SemaphoreType.DMA((N_LAYERS,)),
            pltpu.SemaphoreType.DMA((N_LAYERS,)),
        ],
        compiler_params=pltpu.CompilerParams(
            collective_id=0,
            vmem_limit_bytes=60 * 1024 * 1024,
        ),
    )(x, Win0, Wout0, Win1, Wout1, Win2, Wout2)


# device time: 76305 ns/iter; 2.0811x vs baseline; 1.2548x over previous
import jax
import jax.numpy as jnp
from jax import lax
from jax.experimental import pallas as pl
from jax.experimental.pallas import tpu as pltpu

N_LAYERS = 3
NC = 8
CH = 512
NGC = N_LAYERS * NC
LAG = 2
WSLOTS = 5
KO = 4
CW = 512


def _body(
    x_ref, win0, wout0, win1, wout1, win2, wout2, out_ref,
    wa, wb, h_send, h_recv, o_send, o_send_bf, o_recv_bf, x_buf,
    wa_sem, wb_sem, hs_sems, hr_sems, os_sems, or_sems,
):
    my_x = lax.axis_index("x")
    my_y = lax.axis_index("y")
    y_peer = (my_x, 1 - my_y)
    x_peer = (1 - my_x, my_y)
    wins = [win0, win1, win2]
    wouts = [wout0, wout1, wout2]

    def win_dma(gc):
        l, c = divmod(gc, NC)
        return pltpu.make_async_copy(
            wins[l].at[:, pl.ds(c * CH, CH)],
            wa.at[gc % WSLOTS],
            wa_sem.at[gc % WSLOTS],
        )

    def wout_dma(gc):
        l, c = divmod(gc, NC)
        return pltpu.make_async_copy(
            wouts[l].at[pl.ds(c * CH, CH), :],
            wb.at[gc % WSLOTS],
            wb_sem.at[gc % WSLOTS],
        )

    def h_rdma(gc):
        return pltpu.make_async_remote_copy(
            src_ref=h_send.at[gc],
            dst_ref=h_recv.at[gc],
            send_sem=hs_sems.at[gc],
            recv_sem=hr_sems.at[gc],
            device_id=y_peer,
            device_id_type=pl.DeviceIdType.MESH,
        )

    def o_rdma(l, j):
        return pltpu.make_async_remote_copy(
            src_ref=o_send_bf.at[l, :, pl.ds(j * CW, CW)],
            dst_ref=o_recv_bf.at[l, :, pl.ds(j * CW, CW)],
            send_sem=os_sems.at[l * KO + j],
            recv_sem=or_sems.at[l * KO + j],
            device_id=x_peer,
            device_id_type=pl.DeviceIdType.MESH,
        )

    for s0 in range(WSLOTS):
        win_dma(s0).start()
        wout_dma(s0).start()

    barrier = pltpu.get_barrier_semaphore()
    for t in (y_peer, x_peer):
        pl.semaphore_signal(
            barrier, inc=1, device_id=t, device_id_type=pl.DeviceIdType.MESH
        )
    pl.semaphore_wait(barrier, 2)

    def produce(gc):
        l, c = divmod(gc, NC)
        win_dma(gc).wait()
        w = wa[gc % WSLOTS]
        if l == 0:
            h = jnp.dot(x_ref[...], w, preferred_element_type=jnp.float32)
        elif c == 0:
            h = None
            for j in range(KO):
                o_rdma(l - 1, j).wait_recv()
                sl = slice(j * CW, (j + 1) * CW)
                xj = o_send[l - 1, :, sl] + o_recv_bf[l - 1, :, sl].astype(
                    jnp.float32
                )
                x_buf[:, sl] = xj
                pp = jnp.dot(xj, w[sl, :], preferred_element_type=jnp.float32)
                h = pp if h is None else h + pp
        else:
            h = jnp.dot(x_buf[...], w, preferred_element_type=jnp.float32)
        h_send[gc] = h.astype(jnp.bfloat16)
        h_rdma(gc).start()
        if gc + WSLOTS < NGC:
            win_dma(gc + WSLOTS).start()

    def consume(gc):
        l, c = divmod(gc, NC)
        wout_dma(gc).wait()
        h_rdma(gc).wait_recv()
        hc = jnp.maximum(
            h_send[gc].astype(jnp.float32) + h_recv[gc].astype(jnp.float32), 0.0
        )
        inc = jnp.dot(hc, wb[gc % WSLOTS], preferred_element_type=jnp.float32)
        if c == 0:
            o_send[l] = inc
        else:
            o_send[l] = o_send[l] + inc
        if gc + WSLOTS < NGC:
            wout_dma(gc + WSLOTS).start()

    for l in range(N_LAYERS):
        base = l * NC
        for c in range(LAG):
            produce(base + c)
        for c in range(NC):
            if c + LAG < NC:
                produce(base + c + LAG)
            consume(base + c)
        o_send_bf[l] = o_send[l].astype(jnp.bfloat16)
        for j in range(KO):
            o_rdma(l, j).start()

    for j in range(KO):
        o_rdma(N_LAYERS - 1, j).wait_recv()
        sl = slice(j * CW, (j + 1) * CW)
        out_ref[:, sl] = o_send[N_LAYERS - 1, :, sl] + o_recv_bf[
            N_LAYERS - 1, :, sl
        ].astype(jnp.float32)

    for gc in range(NGC):
        h_rdma(gc).wait_send()
    for l in range(N_LAYERS):
        for j in range(KO):
            o_rdma(l, j).wait_send()


def kernel(x, Win0, Wout0, Win1, Wout1, Win2, Wout2):
    m, k = x.shape
    any_spec = pl.BlockSpec(memory_space=pl.ANY)
    return pl.pallas_call(
        _body,
        out_shape=jax.ShapeDtypeStruct((m, k), jnp.float32),
        in_specs=[pl.BlockSpec(memory_space=pltpu.VMEM)] + [any_spec] * 6,
        out_specs=pl.BlockSpec(memory_space=pltpu.VMEM),
        scratch_shapes=[
            pltpu.VMEM((WSLOTS, k, CH), jnp.float32),
            pltpu.VMEM((WSLOTS, CH, k), jnp.float32),
            pltpu.VMEM((NGC, m, CH), jnp.bfloat16),
            pltpu.VMEM((NGC, m, CH), jnp.bfloat16),
            pltpu.VMEM((N_LAYERS, m, k), jnp.float32),
            pltpu.VMEM((N_LAYERS, m, k), jnp.bfloat16),
            pltpu.VMEM((N_LAYERS, m, k), jnp.bfloat16),
            pltpu.VMEM((m, k), jnp.float32),
            pltpu.SemaphoreType.DMA((WSLOTS,)),
            pltpu.SemaphoreType.DMA((WSLOTS,)),
            pltpu.SemaphoreType.DMA((NGC,)),
            pltpu.SemaphoreType.DMA((NGC,)),
            pltpu.SemaphoreType.DMA((N_LAYERS * KO,)),
            pltpu.SemaphoreType.DMA((N_LAYERS * KO,)),
        ],
        compiler_params=pltpu.CompilerParams(
            collective_id=0,
            vmem_limit_bytes=60 * 1024 * 1024,
        ),
    )(x, Win0, Wout0, Win1, Wout1, Win2, Wout2)
